# speedup vs baseline: 1.3235x; 1.3235x over previous
import functools

import jax
import jax.numpy as jnp
from jax import lax
from jax.experimental import pallas as pl
from jax.experimental.pallas import tpu as pltpu
from jax.experimental.pallas import tpu_sc as plsc

_VOCAB = 30522
_D = 768
_N = 4096 * 50

_NC = 2
_NS = 16
_NW = _NC * _NS
_ROWS_PER_W = _N // _NW
_CHUNK = 128
_NCHUNKS = _ROWS_PER_W // _CHUNK


def _gather_body(table_hbm, idx_hbm, out_hbm, idx_v, rows_v, sem):
    wid = lax.axis_index("s") * _NC + lax.axis_index("c")
    base = wid * _ROWS_PER_W
    pltpu.sync_copy(idx_hbm.at[pl.ds(base, _ROWS_PER_W)], idx_v)

    def chunk(i, carry):
        off = i * _CHUNK
        pltpu.async_copy(
            table_hbm.at[idx_v.at[pl.ds(off, _CHUNK)]], rows_v, sem
        ).wait()
        pltpu.sync_copy(rows_v, out_hbm.at[pl.ds(base + off, _CHUNK)])
        return carry

    lax.fori_loop(0, _NCHUNKS, chunk, 0)


@jax.jit
def _gather(table, idx):
    mesh = plsc.VectorSubcoreMesh(core_axis_name="c", subcore_axis_name="s")
    f = functools.partial(
        pl.kernel,
        out_type=jax.ShapeDtypeStruct((_N, _D), jnp.float32),
        mesh=mesh,
        scratch_types=[
            pltpu.VMEM((_ROWS_PER_W,), jnp.int32),
            pltpu.VMEM((_CHUNK, _D), jnp.float32),
            pltpu.SemaphoreType.DMA,
        ],
    )(_gather_body)
    return f(table, idx)


def kernel(inputs, table):
    idx = inputs.reshape(_N).astype(jnp.int32)
    out = _gather(table, idx)
    return out.reshape(inputs.shape[0], inputs.shape[1], _D)


# speedup vs baseline: 1.3339x; 1.0078x over previous
import functools

import jax
import jax.numpy as jnp
from jax import lax
from jax.experimental import pallas as pl
from jax.experimental.pallas import tpu as pltpu
from jax.experimental.pallas import tpu_sc as plsc

_VOCAB = 30522
_D = 768
_N = 4096 * 50

_NC = 2
_NS = 16
_NW = _NC * _NS
_ROWS_PER_W = _N // _NW
_CHUNK = 64
_NCHUNKS = _ROWS_PER_W // _CHUNK
_NPAIRS = _NCHUNKS // 2


def _gather_body(table_hbm, idx_hbm, out_hbm, idx_v, rows0, rows1, sem0, sem1):
    wid = lax.axis_index("s") * _NC + lax.axis_index("c")
    base = wid * _ROWS_PER_W
    pltpu.sync_copy(idx_hbm.at[pl.ds(base, _ROWS_PER_W)], idx_v)

    bufs = ((rows0, sem0), (rows1, sem1))

    def start(i, rows, sem):
        pltpu.async_copy(
            table_hbm.at[idx_v.at[pl.ds(i * _CHUNK, _CHUNK)]], rows, sem
        )

    def wait(i, rows, sem):
        pltpu.make_async_copy(
            table_hbm.at[idx_v.at[pl.ds(i * _CHUNK, _CHUNK)]], rows, sem
        ).wait()

    for b, (rows, sem) in enumerate(bufs):
        start(b, rows, sem)

    def pair(g, carry):
        for b, (rows, sem) in enumerate(bufs):
            i = 2 * g + b
            wait(i, rows, sem)
            pltpu.sync_copy(rows, out_hbm.at[pl.ds(base + i * _CHUNK, _CHUNK)])
            start(i + 2, rows, sem)
        return carry

    lax.fori_loop(0, _NPAIRS - 1, pair, 0)

    for b, (rows, sem) in enumerate(bufs):
        i = _NCHUNKS - 2 + b
        wait(i, rows, sem)
        pltpu.sync_copy(rows, out_hbm.at[pl.ds(base + i * _CHUNK, _CHUNK)])


@jax.jit
def _gather(table, idx):
    mesh = plsc.VectorSubcoreMesh(core_axis_name="c", subcore_axis_name="s")
    f = functools.partial(
        pl.kernel,
        out_type=jax.ShapeDtypeStruct((_N, _D), jnp.float32),
        mesh=mesh,
        scratch_types=[
            pltpu.VMEM((_ROWS_PER_W,), jnp.int32),
            pltpu.VMEM((_CHUNK, _D), jnp.float32),
            pltpu.VMEM((_CHUNK, _D), jnp.float32),
            pltpu.SemaphoreType.DMA,
            pltpu.SemaphoreType.DMA,
        ],
    )(_gather_body)
    return f(table, idx)


def kernel(inputs, table):
    idx = inputs.reshape(_N).astype(jnp.int32)
    out = _gather(table, idx)
    return out.reshape(inputs.shape[0], inputs.shape[1], _D)
